# baseline (speedup 1.0000x reference)
import jax
import jax.numpy as jnp
from jax.experimental import pallas as pl
from jax.experimental.pallas import tpu as pltpu

_NUM_NEIGHBORS = 8
_BETA = 0.5
_MIN_MARGIN = 0.05
_NUM_HARD_NEG = 4
_ROW_TILE = 256


def _loss_body(s_ref, t_ref, c_ref, o_ref, sn_ref, tn_ref):
    r = pl.program_id(1)
    R = _ROW_TILE
    N = s_ref.shape[1]
    base = r * R

    @pl.when(r == 0)
    def _():
        def _norm(x):
            inv = 1.0 / jnp.maximum(
                jnp.sqrt(jnp.sum(x * x, axis=1, keepdims=True)), 1e-12)
            return x * inv
        sn_ref[...] = _norm(s_ref[0])
        tn_ref[...] = _norm(t_ref[0])

    s_n = sn_ref[...]
    t_n = tn_ref[...]
    s_rows = sn_ref[pl.ds(base, R), :]
    t_rows = tn_ref[pl.ds(base, R), :]
    c_all = c_ref[0]
    c_rows = c_ref[0, pl.ds(base, R), :]

    dn = (((1,), (1,)), ((), ()))
    ssim = jax.lax.dot_general(s_rows, s_n, dn, preferred_element_type=jnp.float32)
    tsim = jax.lax.dot_general(t_rows, t_n, dn, preferred_element_type=jnp.float32)

    cross = jax.lax.dot_general(c_rows, c_all, dn, preferred_element_type=jnp.float32)
    c2_rows = jnp.sum(c_rows * c_rows, axis=1, keepdims=True)
    c2_all = jnp.transpose(jnp.sum(c_all * c_all, axis=1, keepdims=True))
    d2 = c2_rows + c2_all - 2.0 * cross

    inf = jnp.float32(jnp.inf)

    m = jnp.min(d2, axis=1, keepdims=True)
    m1 = m
    for _ in range(_NUM_NEIGHBORS):
        m = jnp.min(jnp.where(d2 > m, d2, inf), axis=1, keepdims=True)
    knn = d2 <= m
    pos_mask = knn & (d2 > m1)

    neg = jnp.where(knn, -inf, tsim)
    g = jnp.max(neg, axis=1, keepdims=True)
    for _ in range(_NUM_HARD_NEG - 1):
        g = jnp.max(jnp.where(neg < g, neg, -inf), axis=1, keepdims=True)
    neg_mask = neg >= g

    zero = jnp.float32(0.0)
    inv_p = jnp.float32(1.0 / _NUM_NEIGHBORS)
    inv_n = jnp.float32(1.0 / _NUM_HARD_NEG)

    d = jnp.abs(jnp.where(pos_mask, ssim - tsim, zero))
    sl1 = jnp.where(d < _BETA, d * d, d - 0.5 * _BETA)

    def _rowsum(x, mask):
        return jnp.sum(jnp.where(mask, x, zero), axis=1, keepdims=True)

    pos_loss = jnp.sum(sl1, axis=1, keepdims=True) * inv_p
    s_pos = _rowsum(ssim, pos_mask) * inv_p
    t_pos = _rowsum(tsim, pos_mask) * inv_p
    s_neg = _rowsum(ssim, neg_mask) * inv_n
    t_neg = _rowsum(tsim, neg_mask) * inv_n

    target = jnp.maximum(t_pos - t_neg, jnp.float32(_MIN_MARGIN))
    margin_loss = jnp.maximum(target - (s_pos - s_neg), zero)
    per_patch = pos_loss + margin_loss
    o_ref[0, 0, pl.ds(base, R)] = per_patch.reshape((R,))


def kernel(student_emb, teacher_emb, centers):
    B, N, D = student_emb.shape
    c_pad = jnp.pad(centers, ((0, 0), (0, 0), (0, 8 - centers.shape[-1])))
    per_patch = pl.pallas_call(
        _loss_body,
        grid=(B, N // _ROW_TILE),
        in_specs=[
            pl.BlockSpec((1, N, D), lambda b, r: (b, 0, 0)),
            pl.BlockSpec((1, N, D), lambda b, r: (b, 0, 0)),
            pl.BlockSpec((1, N, 8), lambda b, r: (b, 0, 0)),
        ],
        out_specs=pl.BlockSpec((1, 1, N), lambda b, r: (b, 0, 0)),
        out_shape=jax.ShapeDtypeStruct((B, 1, N), jnp.float32),
        scratch_shapes=[
            pltpu.VMEM((N, D), jnp.float32),
            pltpu.VMEM((N, D), jnp.float32),
        ],
    )(student_emb, teacher_emb, c_pad)
    return per_patch.mean()


# speedup vs baseline: 1.0633x; 1.0633x over previous
import jax
import jax.numpy as jnp
from jax.experimental import pallas as pl
from jax.experimental.pallas import tpu as pltpu

_NUM_NEIGHBORS = 8
_BETA = 0.5
_MIN_MARGIN = 0.05
_NUM_HARD_NEG = 4
_ROW_TILE = 512


def _loss_body(s_ref, t_ref, c_ref, o_ref, sn_ref, tn_ref):
    r = pl.program_id(1)
    R = _ROW_TILE
    N = s_ref.shape[1]
    base = r * R

    @pl.when(r == 0)
    def _():
        def _norm(x):
            inv = 1.0 / jnp.maximum(
                jnp.sqrt(jnp.sum(x * x, axis=1, keepdims=True)), 1e-12)
            return x * inv
        sn_ref[...] = _norm(s_ref[0])
        tn_ref[...] = _norm(t_ref[0])

    s_n = sn_ref[...]
    t_n = tn_ref[...]
    s_rows = sn_ref[pl.ds(base, R), :]
    t_rows = tn_ref[pl.ds(base, R), :]
    c_all = c_ref[0]
    c_rows = c_ref[0, pl.ds(base, R), :]

    dn = (((1,), (1,)), ((), ()))
    ssim = jax.lax.dot_general(s_rows, s_n, dn, preferred_element_type=jnp.float32)
    tsim = jax.lax.dot_general(t_rows, t_n, dn, preferred_element_type=jnp.float32)

    cross = jax.lax.dot_general(c_rows, c_all, dn, preferred_element_type=jnp.float32)
    c2_rows = jnp.sum(c_rows * c_rows, axis=1, keepdims=True)
    c2_all = jnp.transpose(jnp.sum(c_all * c_all, axis=1, keepdims=True))
    d2 = c2_rows + c2_all - 2.0 * cross

    inf = jnp.float32(jnp.inf)

    m = jnp.min(d2, axis=1, keepdims=True)
    m1 = m
    for _ in range(_NUM_NEIGHBORS):
        m = jnp.min(jnp.where(d2 > m, d2, inf), axis=1, keepdims=True)
    knn = d2 <= m
    pos_mask = knn & (d2 > m1)

    neg = jnp.where(knn, -inf, tsim)
    g = jnp.max(neg, axis=1, keepdims=True)
    for _ in range(_NUM_HARD_NEG - 1):
        g = jnp.max(jnp.where(neg < g, neg, -inf), axis=1, keepdims=True)
    neg_mask = neg >= g

    zero = jnp.float32(0.0)
    inv_p = jnp.float32(1.0 / _NUM_NEIGHBORS)
    inv_n = jnp.float32(1.0 / _NUM_HARD_NEG)

    d = jnp.abs(jnp.where(pos_mask, ssim - tsim, zero))
    sl1 = jnp.where(d < _BETA, d * d, d - 0.5 * _BETA)

    def _rowsum(x, mask):
        return jnp.sum(jnp.where(mask, x, zero), axis=1, keepdims=True)

    pos_loss = jnp.sum(sl1, axis=1, keepdims=True) * inv_p
    s_pos = _rowsum(ssim, pos_mask) * inv_p
    t_pos = _rowsum(tsim, pos_mask) * inv_p
    s_neg = _rowsum(ssim, neg_mask) * inv_n
    t_neg = _rowsum(tsim, neg_mask) * inv_n

    target = jnp.maximum(t_pos - t_neg, jnp.float32(_MIN_MARGIN))
    margin_loss = jnp.maximum(target - (s_pos - s_neg), zero)
    per_patch = pos_loss + margin_loss
    o_ref[0, 0, pl.ds(base, R)] = per_patch.reshape((R,))


def kernel(student_emb, teacher_emb, centers):
    B, N, D = student_emb.shape
    c_pad = jnp.pad(centers, ((0, 0), (0, 0), (0, 8 - centers.shape[-1])))
    per_patch = pl.pallas_call(
        _loss_body,
        grid=(B, N // _ROW_TILE),
        in_specs=[
            pl.BlockSpec((1, N, D), lambda b, r: (b, 0, 0)),
            pl.BlockSpec((1, N, D), lambda b, r: (b, 0, 0)),
            pl.BlockSpec((1, N, 8), lambda b, r: (b, 0, 0)),
        ],
        out_specs=pl.BlockSpec((1, 1, N), lambda b, r: (b, 0, 0)),
        out_shape=jax.ShapeDtypeStruct((B, 1, N), jnp.float32),
        scratch_shapes=[
            pltpu.VMEM((N, D), jnp.float32),
            pltpu.VMEM((N, D), jnp.float32),
        ],
    )(student_emb, teacher_emb, c_pad)
    return per_patch.mean()


# speedup vs baseline: 1.1047x; 1.0390x over previous
import jax
import jax.numpy as jnp
from jax.experimental import pallas as pl
from jax.experimental.pallas import tpu as pltpu

_NUM_NEIGHBORS = 8
_BETA = 0.5
_MIN_MARGIN = 0.05
_NUM_HARD_NEG = 4
_ROW_TILE = 1024


def _loss_body(s_ref, t_ref, c_ref, o_ref, sn_ref, tn_ref):
    r = pl.program_id(1)
    R = _ROW_TILE
    N = s_ref.shape[1]
    base = r * R

    @pl.when(r == 0)
    def _():
        def _norm(x):
            inv = 1.0 / jnp.maximum(
                jnp.sqrt(jnp.sum(x * x, axis=1, keepdims=True)), 1e-12)
            return x * inv
        sn_ref[...] = _norm(s_ref[0])
        tn_ref[...] = _norm(t_ref[0])

    s_n = sn_ref[...]
    t_n = tn_ref[...]
    s_rows = sn_ref[pl.ds(base, R), :]
    t_rows = tn_ref[pl.ds(base, R), :]
    c_all = c_ref[0]
    c_rows = c_ref[0, pl.ds(base, R), :]

    dn = (((1,), (1,)), ((), ()))
    ssim = jax.lax.dot_general(s_rows, s_n, dn, preferred_element_type=jnp.float32)
    tsim = jax.lax.dot_general(t_rows, t_n, dn, preferred_element_type=jnp.float32)

    cross = jax.lax.dot_general(c_rows, c_all, dn, preferred_element_type=jnp.float32)
    c2_rows = jnp.sum(c_rows * c_rows, axis=1, keepdims=True)
    c2_all = jnp.transpose(jnp.sum(c_all * c_all, axis=1, keepdims=True))
    d2 = c2_rows + c2_all - 2.0 * cross

    inf = jnp.float32(jnp.inf)

    m = jnp.min(d2, axis=1, keepdims=True)
    m1 = m
    for _ in range(_NUM_NEIGHBORS):
        m = jnp.min(jnp.where(d2 > m, d2, inf), axis=1, keepdims=True)
    knn = d2 <= m
    pos_mask = knn & (d2 > m1)

    neg = jnp.where(knn, -inf, tsim)
    g = jnp.max(neg, axis=1, keepdims=True)
    for _ in range(_NUM_HARD_NEG - 1):
        g = jnp.max(jnp.where(neg < g, neg, -inf), axis=1, keepdims=True)
    neg_mask = neg >= g

    zero = jnp.float32(0.0)
    inv_p = jnp.float32(1.0 / _NUM_NEIGHBORS)
    inv_n = jnp.float32(1.0 / _NUM_HARD_NEG)

    d = jnp.abs(jnp.where(pos_mask, ssim - tsim, zero))
    sl1 = jnp.where(d < _BETA, d * d, d - 0.5 * _BETA)

    def _rowsum(x, mask):
        return jnp.sum(jnp.where(mask, x, zero), axis=1, keepdims=True)

    pos_loss = jnp.sum(sl1, axis=1, keepdims=True) * inv_p
    s_pos = _rowsum(ssim, pos_mask) * inv_p
    t_pos = _rowsum(tsim, pos_mask) * inv_p
    s_neg = _rowsum(ssim, neg_mask) * inv_n
    t_neg = _rowsum(tsim, neg_mask) * inv_n

    target = jnp.maximum(t_pos - t_neg, jnp.float32(_MIN_MARGIN))
    margin_loss = jnp.maximum(target - (s_pos - s_neg), zero)
    per_patch = pos_loss + margin_loss
    o_ref[0, 0, pl.ds(base, R)] = per_patch.reshape((R,))


def kernel(student_emb, teacher_emb, centers):
    B, N, D = student_emb.shape
    c_pad = jnp.pad(centers, ((0, 0), (0, 0), (0, 8 - centers.shape[-1])))
    per_patch = pl.pallas_call(
        _loss_body,
        grid=(B, N // _ROW_TILE),
        in_specs=[
            pl.BlockSpec((1, N, D), lambda b, r: (b, 0, 0)),
            pl.BlockSpec((1, N, D), lambda b, r: (b, 0, 0)),
            pl.BlockSpec((1, N, 8), lambda b, r: (b, 0, 0)),
        ],
        out_specs=pl.BlockSpec((1, 1, N), lambda b, r: (b, 0, 0)),
        out_shape=jax.ShapeDtypeStruct((B, 1, N), jnp.float32),
        scratch_shapes=[
            pltpu.VMEM((N, D), jnp.float32),
            pltpu.VMEM((N, D), jnp.float32),
        ],
    )(student_emb, teacher_emb, c_pad)
    return per_patch.mean()


# speedup vs baseline: 1.1079x; 1.0029x over previous
import jax
import jax.numpy as jnp
from jax.experimental import pallas as pl

_NUM_NEIGHBORS = 8
_BETA = 0.5
_MIN_MARGIN = 0.05
_NUM_HARD_NEG = 4


def _loss_body(s_ref, t_ref, c_ref, o_ref):
    N = s_ref.shape[1]

    def _norm(x):
        inv = 1.0 / jnp.maximum(
            jnp.sqrt(jnp.sum(x * x, axis=1, keepdims=True)), 1e-12)
        return x * inv

    s_n = _norm(s_ref[0])
    t_n = _norm(t_ref[0])
    c_all = c_ref[0]

    dn = (((1,), (1,)), ((), ()))
    ssim = jax.lax.dot_general(s_n, s_n, dn, preferred_element_type=jnp.float32)
    tsim = jax.lax.dot_general(t_n, t_n, dn, preferred_element_type=jnp.float32)

    cross = jax.lax.dot_general(c_all, c_all, dn, preferred_element_type=jnp.float32)
    c2 = jnp.sum(c_all * c_all, axis=1, keepdims=True)
    d2 = c2 + jnp.transpose(c2) - 2.0 * cross

    inf = jnp.float32(jnp.inf)

    m = jnp.min(d2, axis=1, keepdims=True)
    m1 = m
    for _ in range(_NUM_NEIGHBORS):
        m = jnp.min(jnp.where(d2 > m, d2, inf), axis=1, keepdims=True)
    knn = d2 <= m
    pos_mask = knn & (d2 > m1)

    neg = jnp.where(knn, -inf, tsim)
    g = jnp.max(neg, axis=1, keepdims=True)
    for _ in range(_NUM_HARD_NEG - 1):
        g = jnp.max(jnp.where(neg < g, neg, -inf), axis=1, keepdims=True)
    neg_mask = neg >= g

    zero = jnp.float32(0.0)
    inv_p = jnp.float32(1.0 / _NUM_NEIGHBORS)
    inv_n = jnp.float32(1.0 / _NUM_HARD_NEG)

    d = jnp.abs(jnp.where(pos_mask, ssim - tsim, zero))
    sl1 = jnp.where(d < _BETA, d * d, d - 0.5 * _BETA)

    def _rowsum(x, mask):
        return jnp.sum(jnp.where(mask, x, zero), axis=1, keepdims=True)

    pos_loss = jnp.sum(sl1, axis=1, keepdims=True) * inv_p
    s_pos = _rowsum(ssim, pos_mask) * inv_p
    t_pos = _rowsum(tsim, pos_mask) * inv_p
    s_neg = _rowsum(ssim, neg_mask) * inv_n
    t_neg = _rowsum(tsim, neg_mask) * inv_n

    target = jnp.maximum(t_pos - t_neg, jnp.float32(_MIN_MARGIN))
    margin_loss = jnp.maximum(target - (s_pos - s_neg), zero)
    per_patch = pos_loss + margin_loss
    o_ref[0, 0, :] = per_patch.reshape((N,))


def kernel(student_emb, teacher_emb, centers):
    B, N, D = student_emb.shape
    c_pad = jnp.pad(centers, ((0, 0), (0, 0), (0, 8 - centers.shape[-1])))
    per_patch = pl.pallas_call(
        _loss_body,
        grid=(B,),
        in_specs=[
            pl.BlockSpec((1, N, D), lambda b: (b, 0, 0)),
            pl.BlockSpec((1, N, D), lambda b: (b, 0, 0)),
            pl.BlockSpec((1, N, 8), lambda b: (b, 0, 0)),
        ],
        out_specs=pl.BlockSpec((1, 1, N), lambda b: (b, 0, 0)),
        out_shape=jax.ShapeDtypeStruct((B, 1, N), jnp.float32),
    )(student_emb, teacher_emb, c_pad)
    return per_patch.mean()
